# Initial kernel scaffold; baseline (speedup 1.0000x reference)
import functools

import jax
import jax.numpy as jnp
from jax import lax
from jax.experimental import pallas as pl
from jax.experimental.pallas import tpu as pltpu
from jax.experimental.pallas import tpu_sc as plsc

_N = 100000
_E = 6400000
_D = 4
_LANES = 128
_ROWS = _E // _LANES
_NC = 2
_NS = 16
_NW = _NC * _NS
_R = 16
_ZROWS = _N // _NS


def _sc_segment_sum(dst_rows, edge_attr, zeros):
    mesh = plsc.VectorSubcoreMesh(core_axis_name="c", subcore_axis_name="s")

    @functools.partial(
        pl.kernel,
        out_type=jax.ShapeDtypeStruct((_NC, _N, _D), jnp.float32),
        mesh=mesh,
        scratch_types=[
            pltpu.VMEM((_R, _LANES), jnp.int32),
            pltpu.VMEM((_R * _LANES, _D), jnp.float32),
            pltpu.VMEM_SHARED((_N, _D), jnp.float32),
            pltpu.SemaphoreType.DMA,
        ],
    )
    def k(dst_hbm, attr_hbm, zeros_hbm, out_hbm, idx_v, data_v, acc, sem):
        c = lax.axis_index("c")
        s = lax.axis_index("s")
        gw = c * _NS + s

        pltpu.sync_copy(zeros_hbm.at[pl.ds(s * _ZROWS, _ZROWS), :],
                        acc.at[pl.ds(s * _ZROWS, _ZROWS), :])
        plsc.subcore_barrier()

        base_lo = _ROWS // _NW
        extra = _ROWS - base_lo * _NW
        start = gw * base_lo + jnp.minimum(gw, extra)
        nrows = base_lo + jnp.where(gw < extra, 1, 0)
        nfull = base_lo // _R

        def body(b, carry):
            row0 = start + b * _R
            pltpu.sync_copy(dst_hbm.at[1, pl.ds(row0, _R), :], idx_v)
            pltpu.sync_copy(attr_hbm.at[pl.ds(row0 * _LANES, _R * _LANES), :],
                            data_v)
            cps = []
            for r in range(_R):
                cps.append(pltpu.async_copy(
                    data_v.at[pl.ds(r * _LANES, _LANES), :],
                    acc.at[idx_v.at[r]], sem, add=True))
            for cp in cps:
                cp.wait()
            return carry

        lax.fori_loop(0, nfull, body, 0)

        def rbody(r, carry):
            row = start + nfull * _R + r
            pltpu.sync_copy(dst_hbm.at[1, row], idx_v.at[0])
            pltpu.sync_copy(attr_hbm.at[pl.ds(row * _LANES, _LANES), :],
                            data_v.at[pl.ds(0, _LANES), :])
            pltpu.sync_copy(data_v.at[pl.ds(0, _LANES), :],
                            acc.at[idx_v.at[0]], add=True)
            return carry

        lax.fori_loop(0, nrows - nfull * _R, rbody, 0)
        plsc.subcore_barrier()

        pltpu.sync_copy(acc.at[pl.ds(s * _ZROWS, _ZROWS), :],
                        out_hbm.at[c, pl.ds(s * _ZROWS, _ZROWS), :])

    return k(dst_rows, edge_attr, zeros)


def _tc_finalize(vertex_attr, partials):
    def body(va_ref, p_ref, out_ref):
        a = va_ref[:, 0:1]
        gbar = p_ref[0] + p_ref[1]
        alpha = (1.0 / a) * gbar
        out_ref[...] = jnp.concatenate([va_ref[...], alpha], axis=1)

    return pl.pallas_call(
        body,
        out_shape=jax.ShapeDtypeStruct((_N, 2 + _D), jnp.float32),
    )(vertex_attr, partials)


def kernel(vertex_attr, edgeij_pair, edge_attr, g, batch):
    dst_rows = edgeij_pair.reshape(2, _ROWS, _LANES)
    zeros = jnp.zeros((_N, _D), jnp.float32)
    partials = _sc_segment_sum(dst_rows, edge_attr, zeros)
    return _tc_finalize(vertex_attr, partials)


# speedup vs baseline: 1.6647x; 1.6647x over previous
import functools

import jax
import jax.numpy as jnp
from jax import lax
from jax.experimental import pallas as pl
from jax.experimental.pallas import tpu as pltpu
from jax.experimental.pallas import tpu_sc as plsc

_N = 100000
_E = 6400000
_D = 4
_DP = 8
_LANES = 128
_ROWS = _E // _LANES
_NC = 2
_NS = 16
_NW = _NC * _NS
_R = 8
_UNITS = _ROWS // _R
_UB = _UNITS // _NW
_UX = _UNITS - _UB * _NW
_ZR = 6256
_ZR_LAST = _N - 15 * _ZR


def _sc_segment_sum(dst_rows, edge_attr, zeros):
    mesh = plsc.VectorSubcoreMesh(core_axis_name="c", subcore_axis_name="s")

    @functools.partial(
        pl.kernel,
        out_type=jax.ShapeDtypeStruct((_NC, _N, _DP), jnp.float32),
        mesh=mesh,
        scratch_types=[
            pltpu.VMEM((_R, _LANES), jnp.int32),
            pltpu.VMEM((_R * _LANES, _DP), jnp.float32),
            pltpu.VMEM_SHARED((_N, _DP), jnp.float32),
            pltpu.SemaphoreType.DMA,
        ],
        compiler_params=pltpu.CompilerParams(use_tc_tiling_on_sc=False),
    )
    def k(dst_hbm, attr_hbm, zeros_hbm, out_hbm, idx_v, data_v, acc, sem):
        c = lax.axis_index("c")
        s = lax.axis_index("s")
        gw = c * _NS + s

        pltpu.sync_copy(zeros_hbm.at[pl.ds(0, _R * _LANES), :], data_v)

        @pl.when(s < _NS - 1)
        def _():
            z0 = pl.multiple_of(s * _ZR, 8)
            pltpu.sync_copy(zeros_hbm.at[pl.ds(z0, _ZR), :],
                            acc.at[pl.ds(z0, _ZR), :])

        @pl.when(s == _NS - 1)
        def _():
            pltpu.sync_copy(zeros_hbm.at[pl.ds(15 * _ZR, _ZR_LAST), :],
                            acc.at[pl.ds(15 * _ZR, _ZR_LAST), :])

        plsc.subcore_barrier()

        start_u = gw * _UB + jnp.minimum(gw, _UX)
        nunits = _UB + jnp.where(gw < _UX, 1, 0)

        def body(b, carry):
            row0 = pl.multiple_of((start_u + b) * _R, 8)
            pltpu.sync_copy(dst_hbm.at[1, pl.ds(row0, _R), :], idx_v)
            pltpu.sync_copy(attr_hbm.at[pl.ds(row0 * _LANES, _R * _LANES), :],
                            data_v.at[:, pl.ds(0, _D)])
            cps = []
            for r in range(_R):
                cps.append(pltpu.async_copy(
                    data_v.at[pl.ds(r * _LANES, _LANES), :],
                    acc.at[idx_v.at[r]], sem, add=True))
            for cp in cps:
                cp.wait()
            return carry

        lax.fori_loop(0, nunits, body, 0)
        plsc.subcore_barrier()

        @pl.when(s < _NS - 1)
        def _():
            z0 = pl.multiple_of(s * _ZR, 8)
            pltpu.sync_copy(acc.at[pl.ds(z0, _ZR), :],
                            out_hbm.at[c, pl.ds(z0, _ZR), :])

        @pl.when(s == _NS - 1)
        def _():
            pltpu.sync_copy(acc.at[pl.ds(15 * _ZR, _ZR_LAST), :],
                            out_hbm.at[c, pl.ds(15 * _ZR, _ZR_LAST), :])

    return k(dst_rows, edge_attr, zeros)


_FB = 8192


def _tc_finalize(vertex_attr, partials):
    def body(va_ref, p_ref, out_ref):
        a = va_ref[:, 0:1]
        gbar = p_ref[0, :, 0:_D] + p_ref[1, :, 0:_D]
        alpha = (1.0 / a) * gbar
        out_ref[...] = jnp.concatenate([va_ref[...], alpha], axis=1)

    grid = (_N + _FB - 1) // _FB
    return pl.pallas_call(
        body,
        grid=(grid,),
        in_specs=[
            pl.BlockSpec((_FB, 2), lambda i: (i, 0)),
            pl.BlockSpec((_NC, _FB, _DP), lambda i: (0, i, 0)),
        ],
        out_specs=pl.BlockSpec((_FB, 2 + _D), lambda i: (i, 0)),
        out_shape=jax.ShapeDtypeStruct((_N, 2 + _D), jnp.float32),
    )(vertex_attr, partials)


def kernel(vertex_attr, edgeij_pair, edge_attr, g, batch):
    dst_rows = edgeij_pair.reshape(2, _ROWS, _LANES)
    zeros = jnp.zeros((_N, _DP), jnp.float32)
    partials = _sc_segment_sum(dst_rows, edge_attr, zeros)
    return _tc_finalize(vertex_attr, partials)
